# Initial kernel scaffold; baseline (speedup 1.0000x reference)
import jax
import jax.numpy as jnp
from jax.experimental import pallas as pl


def kernel(x, Wg, w1, w3, w2, ws1, ws3, ws2):
    raise NotImplementedError("write your pallas kernel here")


# speedup vs baseline: 2.1171x; 2.1171x over previous
import functools

import jax
import jax.numpy as jnp
from jax.experimental import pallas as pl
from jax.experimental.pallas import tpu as pltpu


def _router_body(x_ref, wg_ref, xb_ref, i1_ref, i2_ref, m1_ref, m2_ref):
    x = x_ref[...]
    xb_ref[...] = x.astype(jnp.bfloat16)
    s = jax.lax.dot_general(x, wg_ref[...], (((1,), (1,)), ((), ())),
                            preferred_element_type=jnp.float32)
    s = s - jnp.max(s, axis=1, keepdims=True)
    s = jnp.exp(s)
    s = s / jnp.sum(s, axis=1, keepdims=True)
    n_exp = s.shape[1]
    iota = jax.lax.broadcasted_iota(jnp.int32, s.shape, 1)
    m1 = jnp.max(s, axis=1, keepdims=True)
    i1 = jnp.min(jnp.where(s == m1, iota, n_exp), axis=1, keepdims=True)
    s2 = jnp.where(iota == i1, -jnp.inf, s)
    m2 = jnp.max(s2, axis=1, keepdims=True)
    i2 = jnp.min(jnp.where(s2 == m2, iota, n_exp), axis=1, keepdims=True)
    i1_ref[...] = i1
    i2_ref[...] = i2
    m1_ref[...] = m1
    m2_ref[...] = m2


def _router(x, wg, interpret=False):
    t, d = x.shape
    return pl.pallas_call(
        _router_body,
        out_shape=(
            jax.ShapeDtypeStruct((t, d), jnp.bfloat16),
            jax.ShapeDtypeStruct((t, 1), jnp.int32),
            jax.ShapeDtypeStruct((t, 1), jnp.int32),
            jax.ShapeDtypeStruct((t, 1), jnp.float32),
            jax.ShapeDtypeStruct((t, 1), jnp.float32),
        ),
        interpret=interpret,
    )(x, wg)


def _moe_body(n_exp, tb, xb_ref, w1_ref, w3_ref, w2_ref, ws1_ref, ws3_ref,
              ws2_ref, i1_ref, i2_ref, m1_ref, m2_ref, z_ref,
              w1b, w3b, w2b):
    e = pl.program_id(0)
    t = pl.program_id(1)

    @pl.when(t == 0)
    def _():
        @pl.when(e < n_exp)
        def _():
            w1b[...] = w1_ref[0].astype(jnp.bfloat16)
            w3b[...] = w3_ref[0].astype(jnp.bfloat16)
            w2b[...] = w2_ref[0].astype(jnp.bfloat16)

        @pl.when(e == n_exp)
        def _():
            w1b[...] = ws1_ref[...].astype(jnp.bfloat16)
            w3b[...] = ws3_ref[...].astype(jnp.bfloat16)
            w2b[...] = ws2_ref[...].astype(jnp.bfloat16)

    sl = pl.ds(t * tb, tb)
    xt = xb_ref[sl, :]
    h1 = jax.lax.dot_general(xt, w1b[...], (((1,), (1,)), ((), ())),
                             preferred_element_type=jnp.float32)
    h3 = jax.lax.dot_general(xt, w3b[...], (((1,), (1,)), ((), ())),
                             preferred_element_type=jnp.float32)
    g = (h1 * jax.lax.logistic(h1) * h3).astype(jnp.bfloat16)
    y = jax.lax.dot_general(g, w2b[...], (((1,), (1,)), ((), ())),
                            preferred_element_type=jnp.float32)
    wi = jnp.where(e == i1_ref[sl, :], m1_ref[sl, :], 0.0)
    wi = wi + jnp.where(e == i2_ref[sl, :], m2_ref[sl, :], 0.0)
    wi = jnp.where(e == n_exp, 1.0, wi)
    y = y * wi

    @pl.when(e == 0)
    def _():
        z_ref[sl, :] = y

    @pl.when(e > 0)
    def _():
        z_ref[sl, :] = z_ref[sl, :] + y


def _moe(xb, w1, w3, w2, ws1, ws3, ws2, i1, i2, m1, m2, interpret=False):
    t, d = xb.shape
    n_exp, h, _ = w1.shape
    tb = min(1024, t)
    grid = (n_exp + 1, t // tb)
    last = lambda e, tt: (jnp.minimum(e, n_exp - 1), 0, 0)
    const2 = lambda e, tt: (0, 0)
    return pl.pallas_call(
        functools.partial(_moe_body, n_exp, tb),
        grid=grid,
        in_specs=[
            pl.BlockSpec((t, d), const2),
            pl.BlockSpec((1, h, d), last),
            pl.BlockSpec((1, h, d), last),
            pl.BlockSpec((1, d, h), last),
            pl.BlockSpec((h, d), const2),
            pl.BlockSpec((h, d), const2),
            pl.BlockSpec((d, h), const2),
            pl.BlockSpec((t, 1), const2),
            pl.BlockSpec((t, 1), const2),
            pl.BlockSpec((t, 1), const2),
            pl.BlockSpec((t, 1), const2),
        ],
        out_specs=pl.BlockSpec((t, d), const2),
        out_shape=jax.ShapeDtypeStruct((t, d), jnp.float32),
        scratch_shapes=[
            pltpu.VMEM((h, d), jnp.bfloat16),
            pltpu.VMEM((h, d), jnp.bfloat16),
            pltpu.VMEM((d, h), jnp.bfloat16),
        ],
        interpret=interpret,
    )(xb, w1, w3, w2, ws1, ws3, ws2, i1, i2, m1, m2)


def kernel(x, Wg, w1, w3, w2, ws1, ws3, ws2):
    xb, i1, i2, m1, m2 = _router(x, Wg)
    return _moe(xb, w1, w3, w2, ws1, ws3, ws2, i1, i2, m1, m2)
